# baseline (device time: 48924 ns/iter reference)
import jax
import jax.numpy as jnp
from jax import lax
from jax.experimental import pallas as pl
from jax.experimental.pallas import tpu as pltpu

N_DEV = 4
SQ = 1024
SKV = 1024
HQ_LOC = 8
DH = 128
DM = 1024
DLOC = HQ_LOC * DH
CHUNK = SQ // N_DEV
HALF = CHUNK // 2
SCALE = 0.08838834764831843
N_XCHG = 8


def _body(x_ref, wq_ref, k_ref, v_ref, wo_ref, out_ref,
          xs_ref, wqs_ref, ks_ref, vs_ref, wos_ref,
          ctx_ref, partial_ref, comm_ref,
          cp_sems, send_sems, recv_sems):
    my = lax.axis_index("i")

    cp_x = pltpu.make_async_copy(x_ref.at[0], xs_ref, cp_sems.at[0])
    cp_wq = pltpu.make_async_copy(
        wq_ref.at[:, pl.ds(my * DLOC, DLOC)], wqs_ref, cp_sems.at[1])
    cp_wo = pltpu.make_async_copy(
        wo_ref.at[pl.ds(my * DLOC, DLOC), :], wos_ref, cp_sems.at[4])
    for cp in (cp_x, cp_wq, cp_wo):
        cp.start()
    kv_cps = []
    for c in range(4):
        for g in range(4):
            for src, dst, sem in ((k_ref, ks_ref, 2), (v_ref, vs_ref, 3)):
                d = pltpu.make_async_copy(
                    src.at[0, pl.ds((4 * g + c) * 64, 64)],
                    dst.at[c, pl.ds(g * 64, 64)],
                    cp_sems.at[sem])
                d.start()
                kv_cps.append(d)
    cp_x.wait()
    cp_wq.wait()

    q_all = jnp.dot(xs_ref[...].astype(jnp.bfloat16),
                    (wqs_ref[...] * SCALE).astype(jnp.bfloat16),
                    preferred_element_type=jnp.float32)
    qv = q_all.astype(jnp.bfloat16).reshape(4, 4, 64, DLOC)
    for d in kv_cps:
        d.wait()

    for c in range(4):
        qc = qv[:, c].reshape(CHUNK, DLOC)
        kc = ks_ref[c].astype(jnp.bfloat16)
        vc = vs_ref[c].astype(jnp.bfloat16)
        ctxs = []
        for h in range(HQ_LOC):
            q = qc[:, h * DH:(h + 1) * DH]
            k = kc[:, h, :]
            v = vc[:, h, :]
            s = jnp.dot(q, k.T, preferred_element_type=jnp.float32)
            w = jnp.exp(s)
            u = jnp.dot(w.astype(jnp.bfloat16), v,
                        preferred_element_type=jnp.float32)
            ctxs.append(u / jnp.sum(w, axis=-1, keepdims=True))
        ctx_c = jnp.concatenate(ctxs, axis=1)
        ctx_ref[c] = ctx_c.astype(jnp.bfloat16)

    ba = (my % 2 + my // 2) % 2
    bb = my // 2
    yp = my + 1 - 2 * (my % 2)
    xp = 3 - my

    def exch(slot, src, dst_slice, peer):
        rdma = pltpu.make_async_remote_copy(
            src_ref=src,
            dst_ref=dst_slice,
            send_sem=send_sems.at[slot],
            recv_sem=recv_sems.at[slot],
            device_id=(peer,),
            device_id_type=pl.DeviceIdType.MESH,
        )
        rdma.start()
        return rdma

    cp_wo.wait()
    wos = wos_ref[...].astype(jnp.bfloat16)

    def mm_block(g):
        ctg = ctx_ref[:, pl.ds(g * 64, 64), :].reshape(CHUNK, DLOC)
        p = jnp.dot(ctg, wos, preferred_element_type=jnp.float32)
        partial_ref[pl.ds(g, 1)] = p.astype(jnp.bfloat16).reshape(1, CHUNK, DM)

    barrier_sem = pltpu.get_barrier_semaphore()

    mm_block(1 - ba)
    for nbr in (yp, xp):
        pl.semaphore_signal(barrier_sem, inc=1, device_id=(nbr,),
                            device_id_type=pl.DeviceIdType.MESH)
    pl.semaphore_wait(barrier_sem, 2)
    ra = exch(0, partial_ref.at[pl.ds(1 - ba, 1)], comm_ref.at[0:1], yp)
    mm_block(ba)
    mm_block(3 - bb)
    rb = exch(1, partial_ref.at[pl.ds(3 - bb, 1)], comm_ref.at[1:2], xp)
    mm_block(2 + bb)

    ra.wait()
    partial_ref[pl.ds(ba, 1)] = partial_ref[pl.ds(ba, 1)] + comm_ref[0:1]
    ra = exch(2, partial_ref.at[pl.ds(ba, 1), pl.ds(HALF * (1 - bb), HALF)],
              comm_ref.at[2:3, 0:HALF], xp)
    rb.wait()
    partial_ref[pl.ds(2 + bb, 1)] = (
        partial_ref[pl.ds(2 + bb, 1)] + comm_ref[1:2])
    rb = exch(3, partial_ref.at[pl.ds(2 + bb, 1), pl.ds(HALF * (1 - ba), HALF)],
              comm_ref.at[3:4, 0:HALF], yp)

    ra.wait()
    partial_ref[pl.ds(ba, 1), pl.ds(HALF * bb, HALF)] = (
        partial_ref[pl.ds(ba, 1), pl.ds(HALF * bb, HALF)]
        + comm_ref[2:3, 0:HALF])
    ra = exch(4, partial_ref.at[pl.ds(ba, 1), pl.ds(HALF * bb, HALF)],
              comm_ref.at[4:5, 0:HALF], xp)
    rb.wait()
    partial_ref[pl.ds(2 + bb, 1), pl.ds(HALF * ba, HALF)] = (
        partial_ref[pl.ds(2 + bb, 1), pl.ds(HALF * ba, HALF)]
        + comm_ref[3:4, 0:HALF])
    rb = exch(5, partial_ref.at[pl.ds(2 + bb, 1), pl.ds(HALF * ba, HALF)],
              comm_ref.at[5:6, 0:HALF], yp)

    ra.wait()
    partial_ref[pl.ds(ba, 1), pl.ds(HALF * (1 - bb), HALF)] = (
        comm_ref[4:5, 0:HALF])
    ra = exch(6, partial_ref.at[pl.ds(ba, 1)], comm_ref.at[6:7], yp)
    out_ref[0, pl.ds(ba * CHUNK, CHUNK), :] = partial_ref[
        pl.ds(ba, 1)].astype(jnp.float32).reshape(CHUNK, DM)
    rb.wait()
    partial_ref[pl.ds(2 + bb, 1), pl.ds(HALF * (1 - ba), HALF)] = (
        comm_ref[5:6, 0:HALF])
    rb = exch(7, partial_ref.at[pl.ds(2 + bb, 1)], comm_ref.at[7:8], xp)
    out_ref[0, pl.ds((2 + bb) * CHUNK, CHUNK), :] = partial_ref[
        pl.ds(2 + bb, 1)].astype(jnp.float32).reshape(CHUNK, DM)

    ra.wait()
    out_ref[0, pl.ds((1 - ba) * CHUNK, CHUNK), :] = comm_ref[
        6:7].astype(jnp.float32).reshape(CHUNK, DM)
    rb.wait()
    out_ref[0, pl.ds((3 - bb) * CHUNK, CHUNK), :] = comm_ref[
        7:8].astype(jnp.float32).reshape(CHUNK, DM)


def kernel(x, Wq, K_ext, V_ext, Wo):
    return pl.pallas_call(
        _body,
        out_shape=jax.ShapeDtypeStruct((1, SQ, DM), jnp.float32),
        in_specs=[pl.BlockSpec(memory_space=pl.ANY)] * 5,
        out_specs=pl.BlockSpec(memory_space=pltpu.VMEM),
        scratch_shapes=[
            pltpu.VMEM((SQ, DM), jnp.float32),
            pltpu.VMEM((DM, DLOC), jnp.float32),
            pltpu.VMEM((4, CHUNK, HQ_LOC, DH), jnp.float32),
            pltpu.VMEM((4, CHUNK, HQ_LOC, DH), jnp.float32),
            pltpu.VMEM((DLOC, DM), jnp.float32),
            pltpu.VMEM((4, CHUNK, DLOC), jnp.bfloat16),
            pltpu.VMEM((N_DEV, CHUNK, DM), jnp.bfloat16),
            pltpu.VMEM((N_XCHG, CHUNK, DM), jnp.bfloat16),
            pltpu.SemaphoreType.DMA((5,)),
            pltpu.SemaphoreType.DMA((N_XCHG,)),
            pltpu.SemaphoreType.DMA((N_XCHG,)),
        ],
        compiler_params=pltpu.CompilerParams(
            collective_id=0, vmem_limit_bytes=100 * 1024 * 1024),
    )(x, Wq, K_ext, V_ext, Wo)


# device time: 40021 ns/iter; 1.2225x vs baseline; 1.2225x over previous
import jax
import jax.numpy as jnp
from jax import lax
from jax.experimental import pallas as pl
from jax.experimental.pallas import tpu as pltpu

N_DEV = 4
SQ = 1024
SKV = 1024
HQ_LOC = 8
DH = 128
DM = 1024
DLOC = HQ_LOC * DH
CHUNK = SQ // N_DEV
SCALE = 0.08838834764831843
N_SLOT = 32


def _body(x_ref, wq_ref, k_ref, v_ref, wo_ref, out_ref,
          xs_ref, wqs_ref, ks_ref, vs_ref, wos_ref,
          partial_ref, comm_ref,
          cp_sems, send_sems, recv_sems):
    my = lax.axis_index("i")

    cp_wq = pltpu.make_async_copy(
        wq_ref.at[:, pl.ds(my * DLOC, DLOC)], wqs_ref, cp_sems.at[1])
    cp_wo = pltpu.make_async_copy(
        wo_ref.at[pl.ds(my * DLOC, DLOC), :], wos_ref, cp_sems.at[4])
    cp_wq.start()
    cp_wo.start()
    kv_cps = [[] for _ in range(4)]
    for c in range(4):
        for g in range(4):
            for src, dst, sem in ((x_ref, xs_ref, 0), (k_ref, ks_ref, 2),
                                  (v_ref, vs_ref, 3)):
                d = pltpu.make_async_copy(
                    src.at[0, pl.ds((4 * g + c) * 64, 64)],
                    dst.at[c, pl.ds(g * 64, 64)],
                    cp_sems.at[sem])
                d.start()
                kv_cps[c].append(d)
    cp_wq.wait()
    wqb = (wqs_ref[...] * SCALE).astype(jnp.bfloat16)
    cp_wo.wait()
    wos = wos_ref[...].astype(jnp.bfloat16)

    ba = (my % 2 + my // 2) % 2
    bb = my // 2
    yp = my + 1 - 2 * (my % 2)
    xp = 3 - my

    def exch(slot, src, dst_slice, peer):
        rdma = pltpu.make_async_remote_copy(
            src_ref=src,
            dst_ref=dst_slice,
            send_sem=send_sems.at[slot],
            recv_sem=recv_sems.at[slot],
            device_id=(peer,),
            device_id_type=pl.DeviceIdType.MESH,
        )
        rdma.start()
        return rdma

    def rows(c):
        return pl.ds(c * 64, 64)

    def half(c, bit):
        return pl.ds(c * 64 + 32 * bit, 32)

    ras = [None] * 4
    rbs = [None] * 4

    def compute_class(c):
        for d in kv_cps[c]:
            d.wait()
        qc = jnp.dot(xs_ref[c].astype(jnp.bfloat16), wqb,
                     preferred_element_type=jnp.float32).astype(jnp.bfloat16)
        kc = ks_ref[c].astype(jnp.bfloat16)
        vc = vs_ref[c].astype(jnp.bfloat16)
        ctxs = []
        for h in range(HQ_LOC):
            q = qc[:, h * DH:(h + 1) * DH]
            k = kc[:, h, :]
            v = vc[:, h, :]
            s = jnp.dot(q, k.T, preferred_element_type=jnp.float32)
            w = jnp.exp(s)
            u = jnp.dot(w.astype(jnp.bfloat16), v,
                        preferred_element_type=jnp.float32)
            ctxs.append(u / jnp.sum(w, axis=-1, keepdims=True))
        ctx_c = jnp.concatenate(ctxs, axis=1).astype(jnp.bfloat16)
        p = jnp.dot(ctx_c, wos, preferred_element_type=jnp.float32)
        partial_ref[:, rows(c)] = p.astype(jnp.bfloat16).reshape(4, 64, DM)

    def rs1(c):
        s = c * 8
        ras[c] = exch(s, partial_ref.at[pl.ds(1 - ba, 1), rows(c)],
                      comm_ref.at[s:s + 1], yp)
        rbs[c] = exch(s + 1, partial_ref.at[pl.ds(3 - bb, 1), rows(c)],
                      comm_ref.at[s + 1:s + 2], xp)

    def rs2(c):
        s = c * 8
        ras[c].wait()
        partial_ref[pl.ds(ba, 1), rows(c)] = (
            partial_ref[pl.ds(ba, 1), rows(c)] + comm_ref[s:s + 1])
        ras[c] = exch(s + 2, partial_ref.at[pl.ds(ba, 1), half(c, 1 - bb)],
                      comm_ref.at[s + 2:s + 3, 0:32], xp)
        rbs[c].wait()
        partial_ref[pl.ds(2 + bb, 1), rows(c)] = (
            partial_ref[pl.ds(2 + bb, 1), rows(c)] + comm_ref[s + 1:s + 2])
        rbs[c] = exch(s + 3, partial_ref.at[pl.ds(2 + bb, 1), half(c, 1 - ba)],
                      comm_ref.at[s + 3:s + 4, 0:32], yp)

    def ag1(c):
        s = c * 8
        ras[c].wait()
        partial_ref[pl.ds(ba, 1), half(c, bb)] = (
            partial_ref[pl.ds(ba, 1), half(c, bb)]
            + comm_ref[s + 2:s + 3, 0:32])
        ras[c] = exch(s + 4, partial_ref.at[pl.ds(ba, 1), half(c, bb)],
                      comm_ref.at[s + 4:s + 5, 0:32], xp)
        rbs[c].wait()
        partial_ref[pl.ds(2 + bb, 1), half(c, ba)] = (
            partial_ref[pl.ds(2 + bb, 1), half(c, ba)]
            + comm_ref[s + 3:s + 4, 0:32])
        rbs[c] = exch(s + 5, partial_ref.at[pl.ds(2 + bb, 1), half(c, ba)],
                      comm_ref.at[s + 5:s + 6, 0:32], yp)

    def ag2(c):
        s = c * 8
        ras[c].wait()
        partial_ref[pl.ds(ba, 1), half(c, 1 - bb)] = comm_ref[s + 4:s + 5, 0:32]
        ras[c] = exch(s + 6, partial_ref.at[pl.ds(ba, 1), rows(c)],
                      comm_ref.at[s + 6:s + 7], yp)
        rbs[c].wait()
        partial_ref[pl.ds(2 + bb, 1), half(c, 1 - ba)] = (
            comm_ref[s + 5:s + 6, 0:32])
        rbs[c] = exch(s + 7, partial_ref.at[pl.ds(2 + bb, 1), rows(c)],
                      comm_ref.at[s + 7:s + 8], xp)
        out_ref[0, pl.ds(ba * CHUNK + c * 64, 64), :] = partial_ref[
            pl.ds(ba, 1), rows(c)].astype(jnp.float32).reshape(64, DM)
        out_ref[0, pl.ds((2 + bb) * CHUNK + c * 64, 64), :] = partial_ref[
            pl.ds(2 + bb, 1), rows(c)].astype(jnp.float32).reshape(64, DM)

    def fin(c):
        s = c * 8
        ras[c].wait()
        out_ref[0, pl.ds((1 - ba) * CHUNK + c * 64, 64), :] = comm_ref[
            s + 6:s + 7].astype(jnp.float32).reshape(64, DM)
        rbs[c].wait()
        out_ref[0, pl.ds((3 - bb) * CHUNK + c * 64, 64), :] = comm_ref[
            s + 7:s + 8].astype(jnp.float32).reshape(64, DM)

    barrier_sem = pltpu.get_barrier_semaphore()
    stages = [rs1, rs2, ag1, ag2, fin]
    for t in range(8):
        if t < 4:
            compute_class(t)
            if t == 0:
                for nbr in (yp, xp):
                    pl.semaphore_signal(barrier_sem, inc=1, device_id=(nbr,),
                                        device_id_type=pl.DeviceIdType.MESH)
                pl.semaphore_wait(barrier_sem, 2)
        for k, stage in enumerate(stages):
            c = t - k
            if 0 <= c < 4:
                stage(c)


def kernel(x, Wq, K_ext, V_ext, Wo):
    return pl.pallas_call(
        _body,
        out_shape=jax.ShapeDtypeStruct((1, SQ, DM), jnp.float32),
        in_specs=[pl.BlockSpec(memory_space=pl.ANY)] * 5,
        out_specs=pl.BlockSpec(memory_space=pltpu.VMEM),
        scratch_shapes=[
            pltpu.VMEM((4, CHUNK, DM), jnp.float32),
            pltpu.VMEM((DM, DLOC), jnp.float32),
            pltpu.VMEM((4, CHUNK, HQ_LOC, DH), jnp.float32),
            pltpu.VMEM((4, CHUNK, HQ_LOC, DH), jnp.float32),
            pltpu.VMEM((DLOC, DM), jnp.float32),
            pltpu.VMEM((N_DEV, CHUNK, DM), jnp.bfloat16),
            pltpu.VMEM((N_SLOT, 64, DM), jnp.bfloat16),
            pltpu.SemaphoreType.DMA((5,)),
            pltpu.SemaphoreType.DMA((N_SLOT,)),
            pltpu.SemaphoreType.DMA((N_SLOT,)),
        ],
        compiler_params=pltpu.CompilerParams(
            collective_id=0, vmem_limit_bytes=100 * 1024 * 1024),
    )(x, Wq, K_ext, V_ext, Wo)
